# Initial kernel scaffold; baseline (speedup 1.0000x reference)
import jax
import jax.numpy as jnp
from jax.experimental import pallas as pl


def kernel(text, table, W, b):
    raise NotImplementedError("write your pallas kernel here")


# speedup vs baseline: 1.7139x; 1.7139x over previous
import functools

import jax
import jax.numpy as jnp
from jax import lax
from jax.experimental import pallas as pl
from jax.experimental.pallas import tpu as pltpu
from jax.experimental.pallas import tpu_sc as plsc

D = 16
NC, NS = 2, 16
NW = NC * NS
IDXW = 128


def _sc_gather(table, idx2, n_rows):
    per_w = n_rows // NW
    chunk = 2048
    g = chunk // IDXW
    chunks = per_w // chunk
    mesh = plsc.VectorSubcoreMesh(core_axis_name="c", subcore_axis_name="s")

    @functools.partial(
        pl.kernel,
        mesh=mesh,
        out_type=jax.ShapeDtypeStruct((n_rows, D), jnp.float32),
        scratch_types=[
            pltpu.VMEM((g, IDXW), jnp.int32),
            pltpu.VMEM((chunk, D), jnp.float32),
            pltpu.SemaphoreType.DMA,
        ],
        compiler_params=pltpu.CompilerParams(use_tc_tiling_on_sc=False),
    )
    def gather_k(idx_hbm, table_hbm, emb_hbm, idx_v, rows_v, sem):
        wid = lax.axis_index("s") * NC + lax.axis_index("c")
        row0 = wid * per_w

        def chunk_body(i, carry):
            cbase = row0 + i * chunk
            crow = wid * (per_w // IDXW) + i * g
            pltpu.sync_copy(idx_hbm.at[pl.ds(crow, g)], idx_v)
            cps = [
                pltpu.async_copy(
                    table_hbm.at[idx_v.at[j]],
                    rows_v.at[pl.ds(j * IDXW, IDXW)],
                    sem,
                )
                for j in range(g)
            ]
            for cp in cps:
                cp.wait()
            pltpu.sync_copy(rows_v, emb_hbm.at[pl.ds(cbase, chunk)])
            return carry

        lax.fori_loop(0, chunks, chunk_body, 0)

    return gather_k(idx2, table)


def _tc_project(emb2, wbd, bb):
    r = emb2.shape[0]
    blk = 4096

    def body(x_ref, w_ref, b_ref, o_ref):
        o_ref[...] = (
            jnp.dot(x_ref[...], w_ref[...], preferred_element_type=jnp.float32)
            + b_ref[...]
        )

    return pl.pallas_call(
        body,
        grid=(r // blk,),
        in_specs=[
            pl.BlockSpec((blk, 128), lambda i: (i, 0)),
            pl.BlockSpec((128, 128), lambda i: (0, 0)),
            pl.BlockSpec((1, 128), lambda i: (0, 0)),
        ],
        out_specs=pl.BlockSpec((blk, 128), lambda i: (i, 0)),
        out_shape=jax.ShapeDtypeStruct((r, 128), jnp.float32),
    )(emb2, wbd, bb)


def kernel(text, table, W, b):
    bsz, seq = text.shape
    n = bsz * seq
    idx2 = text.reshape(n // IDXW, IDXW).astype(jnp.int32)

    emb_flat = _sc_gather(table, idx2, n)

    wbd = jnp.kron(jnp.eye(8, dtype=W.dtype), W.T)
    bb = jnp.tile(b, 8).reshape(1, 128)
    out_flat = _tc_project(emb_flat.reshape(n // 8, 128), wbd, bb)

    return (
        out_flat.reshape(bsz, seq, D),
        emb_flat.reshape(bsz, seq, D),
    )


# speedup vs baseline: 2.2879x; 1.3349x over previous
import functools

import jax
import jax.numpy as jnp
from jax import lax
from jax.experimental import pallas as pl
from jax.experimental.pallas import tpu as pltpu
from jax.experimental.pallas import tpu_sc as plsc

D = 16
NC, NS = 2, 16
NW = NC * NS
IDXW = 128
CH = 1024
CB = 8192

_SC_PARAMS = pltpu.CompilerParams(
    use_tc_tiling_on_sc=False, needs_layout_passes=False
)


def _tc_prep(table_t, W, bias_bt):
    vocab = table_t.shape[1]
    grid = (vocab + CB - 1) // CB
    rb = CB // 128 * D
    rows = grid * rb

    def body(x_ref, w_ref, b_ref, t_ref, p_ref):
        x = x_ref[...]
        p = jnp.dot(w_ref[...], x, preferred_element_type=jnp.float32)

        def bt(y):
            y3 = y.reshape(D, CB // 128, 128)
            return y3.transpose(1, 0, 2).reshape(rb, 128)

        t_ref[...] = bt(x)
        p_ref[...] = bt(p) + b_ref[...]

    return pl.pallas_call(
        body,
        grid=(grid,),
        in_specs=[
            pl.BlockSpec((D, CB), lambda i: (0, i)),
            pl.BlockSpec((D, D), lambda i: (0, 0)),
            pl.BlockSpec((rb, 128), lambda i: (0, 0)),
        ],
        out_specs=[
            pl.BlockSpec((rb, 128), lambda i: (i, 0)),
            pl.BlockSpec((rb, 128), lambda i: (i, 0)),
        ],
        out_shape=[
            jax.ShapeDtypeStruct((rows, 128), jnp.float32),
            jax.ShapeDtypeStruct((rows, 128), jnp.float32),
        ],
    )(table_t, W, bias_bt)


def _sc_blocktranspose(tbt, pbt):
    rows = tbt.shape[0]
    vbs = rows // D
    per_w = vbs // NW
    vchunk = 6
    chunks = per_w // vchunk
    assert per_w % vchunk == 0
    mesh = plsc.VectorSubcoreMesh(core_axis_name="c", subcore_axis_name="s")
    out_sds = jax.ShapeDtypeStruct((rows * 128 // D, D), jnp.float32)

    @functools.partial(
        pl.kernel,
        mesh=mesh,
        out_type=[out_sds, out_sds],
        scratch_types=[
            pltpu.VMEM((vchunk * D, 128), jnp.float32),
            pltpu.VMEM((vchunk * D, 128), jnp.float32),
            pltpu.VMEM((vchunk * 128, D), jnp.float32),
            pltpu.VMEM((vchunk * 128, D), jnp.float32),
        ],
        compiler_params=_SC_PARAMS,
    )
    def bt_k(tbt_hbm, pbt_hbm, tlin_hbm, plin_hbm, tin_v, pin_v, tout_v, pout_v):
        wid = lax.axis_index("s") * NC + lax.axis_index("c")
        iota16 = lax.broadcasted_iota(jnp.int32, (16,), 0)

        def chunk_body(k, carry):
            vb0 = wid * per_w + k * vchunk
            pltpu.sync_copy(tbt_hbm.at[pl.ds(vb0 * D, vchunk * D)], tin_v)
            pltpu.sync_copy(pbt_hbm.at[pl.ds(vb0 * D, vchunk * D)], pin_v)
            for c in range(vchunk):
                def col_body(vl, carry2):
                    lv = jnp.full((16,), vl, jnp.int32)
                    rv = iota16 + c * D
                    tout_v[c * 128 + vl] = plsc.load_gather(tin_v, [rv, lv])
                    pout_v[c * 128 + vl] = plsc.load_gather(pin_v, [rv, lv])
                    return carry2

                lax.fori_loop(0, 128, col_body, 0)
            pltpu.sync_copy(tout_v, tlin_hbm.at[pl.ds(vb0 * 128, vchunk * 128)])
            pltpu.sync_copy(pout_v, plin_hbm.at[pl.ds(vb0 * 128, vchunk * 128)])
            return carry

        lax.fori_loop(0, chunks, chunk_body, 0)

    return bt_k(tbt, pbt)


def _sc_gather2(idx3, tlin, plin, seq, bsz):
    n = seq * bsz
    bblks = bsz // CH
    chunks = n // CH // NW
    g = CH // IDXW
    mesh = plsc.VectorSubcoreMesh(core_axis_name="c", subcore_axis_name="s")
    out_sds = jax.ShapeDtypeStruct((seq, 2, bsz // 128, 8, 128), jnp.float32)

    @functools.partial(
        pl.kernel,
        mesh=mesh,
        out_type=[out_sds, out_sds],
        scratch_types=[
            pltpu.VMEM((g, IDXW), jnp.int32),
            pltpu.VMEM((CH, D), jnp.float32),
            pltpu.VMEM((CH, D), jnp.float32),
            pltpu.VMEM((2, CH // 128, 8, 128), jnp.float32),
            pltpu.VMEM((2, CH // 128, 8, 128), jnp.float32),
            pltpu.SemaphoreType.DMA,
            pltpu.SemaphoreType.DMA,
        ],
        compiler_params=_SC_PARAMS,
    )
    def gather_k(idx_hbm, tlin_hbm, plin_hbm, emb_hbm, out_hbm,
                 idx_v, erows_v, prows_v, et_v, pt_v, esem, psem):
        wid = lax.axis_index("s") * NC + lax.axis_index("c")
        iota16 = lax.broadcasted_iota(jnp.int32, (16,), 0)

        def chunk_body(k, carry):
            cid = wid * chunks + k
            l = cid // bblks
            bblk = cid % bblks
            pltpu.sync_copy(idx_hbm.at[l, pl.ds(bblk * g, g)], idx_v)
            ecps = [
                pltpu.async_copy(
                    tlin_hbm.at[idx_v.at[j]],
                    erows_v.at[pl.ds(j * IDXW, IDXW)],
                    esem,
                )
                for j in range(g)
            ]
            pcps = [
                pltpu.async_copy(
                    plin_hbm.at[idx_v.at[j]],
                    prows_v.at[pl.ds(j * IDXW, IDXW)],
                    psem,
                )
                for j in range(g)
            ]
            for cp in ecps:
                cp.wait()
            for cp in pcps:
                cp.wait()

            def tr_body(tc, carry2):
                base = tc * 128
                for cg in range(8):
                    ridx = iota16 + (base + cg * 16)
                    for f in range(D):
                        fv = jnp.full((16,), f, jnp.int32)
                        ev = plsc.load_gather(erows_v, [ridx, fv])
                        pv = plsc.load_gather(prows_v, [ridx, fv])
                        et_v[f // 8, tc, f % 8, pl.ds(cg * 16, 16)] = ev
                        pt_v[f // 8, tc, f % 8, pl.ds(cg * 16, 16)] = pv
                return carry2

            lax.fori_loop(0, CH // 128, tr_body, 0)

            for tr in range(2):
                pltpu.sync_copy(
                    et_v.at[tr], emb_hbm.at[l, tr, pl.ds(bblk * (CH // 128), CH // 128)]
                )
                pltpu.sync_copy(
                    pt_v.at[tr], out_hbm.at[l, tr, pl.ds(bblk * (CH // 128), CH // 128)]
                )
            return carry

        lax.fori_loop(0, chunks, chunk_body, 0)

    return gather_k(idx3, tlin, plin)


def kernel(text, table, W, b):
    bsz, seq = text.shape

    bias_bt = jnp.broadcast_to(
        jnp.tile(b, CB // 128)[:, None], (CB // 128 * D, 128)
    )
    tbt, pbt = _tc_prep(table.T, W, bias_bt)
    tlin, plin = _sc_blocktranspose(tbt, pbt)

    idx3 = text.T.astype(jnp.int32).reshape(seq, bsz // 128, 128)

    emb5, out5 = _sc_gather2(idx3, tlin, plin, seq, bsz)

    emb = emb5.transpose((2, 4, 0, 1, 3)).reshape(bsz, seq, D)
    out = out5.transpose((2, 4, 0, 1, 3)).reshape(bsz, seq, D)
    return (out, emb)


# speedup vs baseline: 2.4024x; 1.0501x over previous
import functools

import jax
import jax.numpy as jnp
from jax import lax
from jax.experimental import pallas as pl
from jax.experimental.pallas import tpu as pltpu
from jax.experimental.pallas import tpu_sc as plsc

D = 16
NC, NS = 2, 16
NW = NC * NS
IDXW = 128
CH = 1024
CB = 8192

_SC_PARAMS = pltpu.CompilerParams(
    use_tc_tiling_on_sc=False, needs_layout_passes=False
)


def _tc_prep(table_t, W, bias_bt):
    vocab = table_t.shape[1]
    grid = (vocab + CB - 1) // CB
    rb = CB // 128 * D
    rows = grid * rb

    def body(x_ref, w_ref, b_ref, t_ref, p_ref):
        x = x_ref[...]
        p = jnp.dot(w_ref[...], x, preferred_element_type=jnp.float32)

        def bt(y):
            y3 = y.reshape(D, CB // 128, 128)
            return y3.transpose(1, 0, 2).reshape(rb, 128)

        t_ref[...] = bt(x)
        p_ref[...] = bt(p) + b_ref[...]

    return pl.pallas_call(
        body,
        grid=(grid,),
        in_specs=[
            pl.BlockSpec((D, CB), lambda i: (0, i)),
            pl.BlockSpec((D, D), lambda i: (0, 0)),
            pl.BlockSpec((rb, 128), lambda i: (0, 0)),
        ],
        out_specs=[
            pl.BlockSpec((rb, 128), lambda i: (i, 0)),
            pl.BlockSpec((rb, 128), lambda i: (i, 0)),
        ],
        out_shape=[
            jax.ShapeDtypeStruct((rows, 128), jnp.float32),
            jax.ShapeDtypeStruct((rows, 128), jnp.float32),
        ],
    )(table_t, W, bias_bt)


def _sc_blocktranspose(tbt, pbt):
    rows = tbt.shape[0]
    vbs = rows // D
    per_w = vbs // NW
    vchunk = 6
    chunks = per_w // vchunk
    assert per_w % vchunk == 0
    mesh = plsc.VectorSubcoreMesh(core_axis_name="c", subcore_axis_name="s")
    out_sds = jax.ShapeDtypeStruct((rows * 128 // D, D), jnp.float32)
    ir = vchunk * D
    orr = vchunk * 128

    @functools.partial(
        pl.kernel,
        mesh=mesh,
        out_type=[out_sds, out_sds],
        scratch_types=[
            pltpu.VMEM((2 * ir, 128), jnp.float32),
            pltpu.VMEM((2 * ir, 128), jnp.float32),
            pltpu.VMEM((2 * orr, D), jnp.float32),
            pltpu.VMEM((2 * orr, D), jnp.float32),
            pltpu.SemaphoreType.DMA,
            pltpu.SemaphoreType.DMA,
            pltpu.SemaphoreType.DMA,
            pltpu.SemaphoreType.DMA,
        ],
        compiler_params=_SC_PARAMS,
    )
    def bt_k(tbt_hbm, pbt_hbm, tlin_hbm, plin_hbm,
             tin_v, pin_v, tout_v, pout_v, isem0, isem1, osem0, osem1):
        wid = lax.axis_index("s") * NC + lax.axis_index("c")
        iota16 = lax.broadcasted_iota(jnp.int32, (16,), 0)
        i128 = iota16 * 128
        isems = (isem0, isem1)
        osems = (osem0, osem1)

        def fire_in(k, buf):
            vb0 = wid * per_w + k * vchunk
            pltpu.async_copy(
                tbt_hbm.at[pl.ds(vb0 * D, ir)], tin_v.at[pl.ds(buf * ir, ir)],
                isems[buf])
            pltpu.async_copy(
                pbt_hbm.at[pl.ds(vb0 * D, ir)], pin_v.at[pl.ds(buf * ir, ir)],
                isems[buf])

        def drain(sem, hbm, vref):
            pltpu.make_async_copy(hbm, vref, sem).wait()

        def process(k, buf):
            drain(isems[buf], tbt_hbm.at[pl.ds(0, ir)],
                  tin_v.at[pl.ds(buf * ir, ir)])
            drain(isems[buf], pbt_hbm.at[pl.ds(0, ir)],
                  pin_v.at[pl.ds(buf * ir, ir)])

            if k >= 2:
                drain(osems[buf], tlin_hbm.at[pl.ds(0, orr)],
                      tout_v.at[pl.ds(buf * orr, orr)])
                drain(osems[buf], plin_hbm.at[pl.ds(0, orr)],
                      pout_v.at[pl.ds(buf * orr, orr)])

            obase = buf * orr

            def col_body(vl, carry2):
                lv = jnp.full((16,), vl, jnp.int32)
                for c in range(vchunk):
                    rv16 = jnp.full((16,), buf * ir + c * D, jnp.int32) + \
                        lax.broadcasted_iota(jnp.int32, (16,), 0)
                    orow = obase + c * 128 + vl
                    tout_v[orow] = plsc.load_gather(tin_v, [rv16, lv])
                    pout_v[orow] = plsc.load_gather(pin_v, [rv16, lv])
                return carry2

            lax.fori_loop(0, 128, col_body, 0)
            vb0 = wid * per_w + k * vchunk
            pltpu.async_copy(tout_v.at[pl.ds(obase, orr)],
                             tlin_hbm.at[pl.ds(vb0 * 128, orr)], osems[buf])
            pltpu.async_copy(pout_v.at[pl.ds(obase, orr)],
                             plin_hbm.at[pl.ds(vb0 * 128, orr)], osems[buf])

        fire_in(0, 0)

        for k in range(chunks):
            buf = k % 2
            knext = min(k + 1, chunks - 1)
            fire_in(knext, 1 - buf)
            process(k, buf)

        for buf in range(2):
            drain(osems[buf], tlin_hbm.at[pl.ds(0, orr)],
                  tout_v.at[pl.ds(buf * orr, orr)])
            drain(osems[buf], plin_hbm.at[pl.ds(0, orr)],
                  pout_v.at[pl.ds(buf * orr, orr)])
        extra = chunks % 2
        drain(isems[extra], tbt_hbm.at[pl.ds(0, ir)],
              tin_v.at[pl.ds(extra * ir, ir)])
        drain(isems[extra], pbt_hbm.at[pl.ds(0, ir)],
              pin_v.at[pl.ds(extra * ir, ir)])

    return bt_k(tbt, pbt)


def _sc_gather2(idx3, tlin, plin, seq, bsz):
    n = seq * bsz
    bblks = bsz // CH
    chunks = n // CH // NW
    g = CH // IDXW
    mesh = plsc.VectorSubcoreMesh(core_axis_name="c", subcore_axis_name="s")
    out_sds = jax.ShapeDtypeStruct((seq, 2, bsz // 128, 8, 128), jnp.float32)

    @functools.partial(
        pl.kernel,
        mesh=mesh,
        out_type=[out_sds, out_sds],
        scratch_types=[
            pltpu.VMEM((g, IDXW), jnp.int32),
            pltpu.VMEM((CH, D), jnp.float32),
            pltpu.VMEM((CH, D), jnp.float32),
            pltpu.VMEM((2, CH // 128, 8, 128), jnp.float32),
            pltpu.VMEM((2, CH // 128, 8, 128), jnp.float32),
            pltpu.SemaphoreType.DMA,
            pltpu.SemaphoreType.DMA,
        ],
        compiler_params=_SC_PARAMS,
    )
    def gather_k(idx_hbm, tlin_hbm, plin_hbm, emb_hbm, out_hbm,
                 idx_v, erows_v, prows_v, et_v, pt_v, esem, psem):
        wid = lax.axis_index("s") * NC + lax.axis_index("c")
        iota16 = lax.broadcasted_iota(jnp.int32, (16,), 0)

        def chunk_body(k, carry):
            cid = wid * chunks + k
            l = cid // bblks
            bblk = cid % bblks
            pltpu.sync_copy(idx_hbm.at[l, pl.ds(bblk * g, g)], idx_v)
            ecps = [
                pltpu.async_copy(
                    tlin_hbm.at[idx_v.at[j]],
                    erows_v.at[pl.ds(j * IDXW, IDXW)],
                    esem,
                )
                for j in range(g)
            ]
            pcps = [
                pltpu.async_copy(
                    plin_hbm.at[idx_v.at[j]],
                    prows_v.at[pl.ds(j * IDXW, IDXW)],
                    psem,
                )
                for j in range(g)
            ]
            for cp in ecps:
                cp.wait()
            for cp in pcps:
                cp.wait()

            def tr_body(tc, carry2):
                base = tc * 128
                for cg in range(8):
                    ridx = iota16 + (base + cg * 16)
                    for f in range(D):
                        fv = jnp.full((16,), f, jnp.int32)
                        ev = plsc.load_gather(erows_v, [ridx, fv])
                        pv = plsc.load_gather(prows_v, [ridx, fv])
                        et_v[f // 8, tc, f % 8, pl.ds(cg * 16, 16)] = ev
                        pt_v[f // 8, tc, f % 8, pl.ds(cg * 16, 16)] = pv
                return carry2

            lax.fori_loop(0, CH // 128, tr_body, 0)

            for tr in range(2):
                pltpu.sync_copy(
                    et_v.at[tr], emb_hbm.at[l, tr, pl.ds(bblk * (CH // 128), CH // 128)]
                )
                pltpu.sync_copy(
                    pt_v.at[tr], out_hbm.at[l, tr, pl.ds(bblk * (CH // 128), CH // 128)]
                )
            return carry

        lax.fori_loop(0, chunks, chunk_body, 0)

    return gather_k(idx3, tlin, plin)


def kernel(text, table, W, b):
    bsz, seq = text.shape

    bias_bt = jnp.broadcast_to(
        jnp.tile(b, CB // 128)[:, None], (CB // 128 * D, 128)
    )
    tbt, pbt = _tc_prep(table.T, W, bias_bt)
    tlin, plin = _sc_blocktranspose(tbt, pbt)

    idx3 = text.T.astype(jnp.int32).reshape(seq, bsz // 128, 128)

    emb5, out5 = _sc_gather2(idx3, tlin, plin, seq, bsz)

    emb = emb5.transpose((2, 4, 0, 1, 3)).reshape(bsz, seq, D)
    out = out5.transpose((2, 4, 0, 1, 3)).reshape(bsz, seq, D)
    return (out, emb)


# speedup vs baseline: 2.4579x; 1.0231x over previous
import functools

import jax
import jax.numpy as jnp
from jax import lax
from jax.experimental import pallas as pl
from jax.experimental.pallas import tpu as pltpu
from jax.experimental.pallas import tpu_sc as plsc

D = 16
NC, NS = 2, 16
NW = NC * NS
IDXW = 128
CH = 1024
CB = 8192

_SC_PARAMS = pltpu.CompilerParams(
    use_tc_tiling_on_sc=False, needs_layout_passes=False
)


def _tc_prep(table_t):
    vocab = table_t.shape[1]
    grid = (vocab + CB - 1) // CB
    rb = CB // 128 * D
    rows = grid * rb

    def body(x_ref, t_ref):
        x = x_ref[...]
        y3 = x.reshape(D, CB // 128, 128)
        t_ref[...] = y3.transpose(1, 0, 2).reshape(rb, 128)

    return pl.pallas_call(
        body,
        grid=(grid,),
        in_specs=[pl.BlockSpec((D, CB), lambda i: (0, i))],
        out_specs=pl.BlockSpec((rb, 128), lambda i: (i, 0)),
        out_shape=jax.ShapeDtypeStruct((rows, 128), jnp.float32),
    )(table_t)


def _sc_blocktranspose(tbt):
    rows = tbt.shape[0]
    vbs = rows // D
    per_w = vbs // NW
    vchunk = 6
    chunks = per_w // vchunk
    assert per_w % vchunk == 0
    mesh = plsc.VectorSubcoreMesh(core_axis_name="c", subcore_axis_name="s")
    out_sds = jax.ShapeDtypeStruct((rows * 128 // D, D), jnp.float32)
    ir = vchunk * D
    orr = vchunk * 128

    @functools.partial(
        pl.kernel,
        mesh=mesh,
        out_type=out_sds,
        scratch_types=[
            pltpu.VMEM((2 * ir, 128), jnp.float32),
            pltpu.VMEM((2 * orr, D), jnp.float32),
            pltpu.SemaphoreType.DMA,
            pltpu.SemaphoreType.DMA,
            pltpu.SemaphoreType.DMA,
            pltpu.SemaphoreType.DMA,
        ],
        compiler_params=_SC_PARAMS,
    )
    def bt_k(tbt_hbm, tlin_hbm, tin_v, tout_v, isem0, isem1, osem0, osem1):
        wid = lax.axis_index("s") * NC + lax.axis_index("c")
        isems = (isem0, isem1)
        osems = (osem0, osem1)

        def fire_in(k, buf):
            vb0 = wid * per_w + k * vchunk
            pltpu.async_copy(
                tbt_hbm.at[pl.ds(vb0 * D, ir)], tin_v.at[pl.ds(buf * ir, ir)],
                isems[buf])

        def drain(sem, hbm, vref):
            pltpu.make_async_copy(hbm, vref, sem).wait()

        def process(k, buf):
            drain(isems[buf], tbt_hbm.at[pl.ds(0, ir)],
                  tin_v.at[pl.ds(buf * ir, ir)])

            if k >= 2:
                drain(osems[buf], tlin_hbm.at[pl.ds(0, orr)],
                      tout_v.at[pl.ds(buf * orr, orr)])

            obase = buf * orr

            def col_body(vl, carry2):
                lv = jnp.full((16,), vl, jnp.int32)
                for c in range(vchunk):
                    rv16 = jnp.full((16,), buf * ir + c * D, jnp.int32) + \
                        lax.broadcasted_iota(jnp.int32, (16,), 0)
                    orow = obase + c * 128 + vl
                    tout_v[orow] = plsc.load_gather(tin_v, [rv16, lv])
                return carry2

            lax.fori_loop(0, 128, col_body, 0)
            vb0 = wid * per_w + k * vchunk
            pltpu.async_copy(tout_v.at[pl.ds(obase, orr)],
                             tlin_hbm.at[pl.ds(vb0 * 128, orr)], osems[buf])

        fire_in(0, 0)

        for k in range(chunks):
            buf = k % 2
            knext = min(k + 1, chunks - 1)
            fire_in(knext, 1 - buf)
            process(k, buf)

        for buf in range(2):
            drain(osems[buf], tlin_hbm.at[pl.ds(0, orr)],
                  tout_v.at[pl.ds(buf * orr, orr)])
        extra = chunks % 2
        drain(isems[extra], tbt_hbm.at[pl.ds(0, ir)],
              tin_v.at[pl.ds(extra * ir, ir)])

    return bt_k(tbt)


def _sc_gather(idx3, tlin, seq, bsz):
    n = seq * bsz
    bblks = bsz // CH
    chunks = n // CH // NW
    g = CH // IDXW
    mesh = plsc.VectorSubcoreMesh(core_axis_name="c", subcore_axis_name="s")
    out_sds = jax.ShapeDtypeStruct((seq, 2, bsz // 128, 8, 128), jnp.float32)

    @functools.partial(
        pl.kernel,
        mesh=mesh,
        out_type=out_sds,
        scratch_types=[
            pltpu.VMEM((g, IDXW), jnp.int32),
            pltpu.VMEM((CH, D), jnp.float32),
            pltpu.VMEM((2, CH // 128, 8, 128), jnp.float32),
            pltpu.SemaphoreType.DMA,
        ],
        compiler_params=_SC_PARAMS,
    )
    def gather_k(idx_hbm, tlin_hbm, emb_hbm, idx_v, erows_v, et_v, esem):
        wid = lax.axis_index("s") * NC + lax.axis_index("c")
        iota16 = lax.broadcasted_iota(jnp.int32, (16,), 0)

        def chunk_body(k, carry):
            cid = wid * chunks + k
            l = cid // bblks
            bblk = cid % bblks
            pltpu.sync_copy(idx_hbm.at[l, pl.ds(bblk * g, g)], idx_v)
            ecps = [
                pltpu.async_copy(
                    tlin_hbm.at[idx_v.at[j]],
                    erows_v.at[pl.ds(j * IDXW, IDXW)],
                    esem,
                )
                for j in range(g)
            ]
            for cp in ecps:
                cp.wait()

            def tr_body(tc, carry2):
                base = tc * 128
                for cg in range(8):
                    ridx = iota16 + (base + cg * 16)
                    for f in range(D):
                        fv = jnp.full((16,), f, jnp.int32)
                        ev = plsc.load_gather(erows_v, [ridx, fv])
                        et_v[f // 8, tc, f % 8, pl.ds(cg * 16, 16)] = ev
                return carry2

            lax.fori_loop(0, CH // 128, tr_body, 0)

            for tr in range(2):
                pltpu.sync_copy(
                    et_v.at[tr],
                    emb_hbm.at[l, tr, pl.ds(bblk * (CH // 128), CH // 128)],
                )
            return carry

        lax.fori_loop(0, chunks, chunk_body, 0)

    return gather_k(idx3, tlin)


def _tc_proj(emb5, W, b2):
    seq, _, nb, _, _ = emb5.shape
    Bk = 16
    grid = (seq, nb // Bk)

    def body(x_ref, w_ref, b_ref, o_ref):
        x = x_ref[0]
        xf = x.transpose(0, 2, 1, 3).reshape(D, Bk * 128)
        y = jnp.dot(w_ref[...], xf, preferred_element_type=jnp.float32)
        y = y + b_ref[...]
        o_ref[0] = y.reshape(2, 8, Bk, 128).transpose(0, 2, 1, 3)

    return pl.pallas_call(
        body,
        grid=grid,
        in_specs=[
            pl.BlockSpec((1, 2, Bk, 8, 128), lambda i, j: (i, 0, j, 0, 0)),
            pl.BlockSpec((D, D), lambda i, j: (0, 0)),
            pl.BlockSpec((D, Bk * 128), lambda i, j: (0, 0)),
        ],
        out_specs=pl.BlockSpec((1, 2, Bk, 8, 128), lambda i, j: (i, 0, j, 0, 0)),
        out_shape=jax.ShapeDtypeStruct(emb5.shape, jnp.float32),
    )(emb5, W, b2)


def kernel(text, table, W, b):
    bsz, seq = text.shape

    tbt = _tc_prep(table.T)
    tlin = _sc_blocktranspose(tbt)

    idx3 = text.T.astype(jnp.int32).reshape(seq, bsz // 128, 128)

    emb5 = _sc_gather(idx3, tlin, seq, bsz)

    b2 = jnp.broadcast_to(b[:, None], (D, 16 * 128))
    out5 = _tc_proj(emb5, W, b2)

    emb = emb5.transpose((2, 4, 0, 1, 3)).reshape(bsz, seq, D)
    out = out5.transpose((2, 4, 0, 1, 3)).reshape(bsz, seq, D)
    return (out, emb)
